# Initial kernel scaffold; baseline (speedup 1.0000x reference)
import functools

import jax
import jax.numpy as jnp
from jax import lax
from jax.experimental import pallas as pl
from jax.experimental.pallas import tpu as pltpu
from jax.experimental.pallas import tpu_sc as plsc

N = 10000
D = 128
EPS = 1e-5

NC = 2
NS = 16
NW = NC * NS

CHUNK = 128
NPAD = 10240
DEG_W = 8
ZROWS = 125

_mesh = plsc.VectorSubcoreMesh(
    core_axis_name="c", subcore_axis_name="s", num_cores=NC, num_subcores=NS
)


def _worker(cid, sid):
    return sid * NC + cid


def _make_deg(E):
    nchunk = E // CHUNK
    base, rem = nchunk // NW, nchunk % NW

    @functools.partial(
        pl.kernel,
        out_type=jax.ShapeDtypeStruct((NC, NPAD, DEG_W), jnp.float32),
        mesh=_mesh,
        scratch_types=[
            pltpu.VMEM((CHUNK,), jnp.int32),
            pltpu.VMEM((CHUNK, DEG_W), jnp.float32),
            pltpu.VMEM_SHARED((NPAD, DEG_W), jnp.float32),
            pltpu.SemaphoreType.DMA,
        ],
    )
    def deg_k(dst_hbm, ones_hbm, zer_hbm, out_hbm, idx_v, ones_v, deg_sp, sem):
        cid = lax.axis_index("c")
        sid = lax.axis_index("s")
        wid = _worker(cid, sid)
        rpt = NPAD // NS
        pltpu.sync_copy(zer_hbm, deg_sp.at[pl.ds(sid * rpt, rpt)])
        pltpu.sync_copy(ones_hbm, ones_v)
        plsc.subcore_barrier()

        def body(t, carry):
            j = wid + NW * t
            pltpu.sync_copy(dst_hbm.at[pl.ds(j * CHUNK, CHUNK)], idx_v)
            pltpu.sync_copy(ones_v, deg_sp.at[idx_v], add=True)
            return carry

        lax.fori_loop(0, base + (wid < rem).astype(jnp.int32), body, 0)
        plsc.subcore_barrier()
        pltpu.sync_copy(
            deg_sp.at[pl.ds(sid * rpt, rpt)],
            out_hbm.at[cid, pl.ds(sid * rpt, rpt)],
        )

    return deg_k


def _make_scatter(E):
    nchunk = E // CHUNK
    base, rem = nchunk // NW, nchunk % NW

    @functools.partial(
        pl.kernel,
        out_type=jax.ShapeDtypeStruct((NC, N, D), jnp.float32),
        mesh=_mesh,
        scratch_types=[
            pltpu.VMEM((CHUNK,), jnp.int32),
            pltpu.VMEM((CHUNK,), jnp.int32),
            pltpu.VMEM((CHUNK, D), jnp.float32),
            pltpu.VMEM_SHARED((N, D), jnp.float32),
            pltpu.SemaphoreType.DMA,
        ],
    )
    def scat_k(hp_hbm, src_hbm, dst_hbm, zer_hbm, out_hbm,
               sidx, didx, rows, acc_sp, sem):
        cid = lax.axis_index("c")
        sid = lax.axis_index("s")
        wid = _worker(cid, sid)
        rpt = N // NS
        for z in range(rpt // ZROWS):
            pltpu.sync_copy(zer_hbm, acc_sp.at[pl.ds(sid * rpt + z * ZROWS, ZROWS)])
        plsc.subcore_barrier()

        def body(t, carry):
            j = wid + NW * t
            pltpu.sync_copy(src_hbm.at[pl.ds(j * CHUNK, CHUNK)], sidx)
            pltpu.sync_copy(dst_hbm.at[pl.ds(j * CHUNK, CHUNK)], didx)
            pltpu.async_copy(hp_hbm.at[sidx], rows, sem).wait()
            pltpu.sync_copy(rows, acc_sp.at[didx], add=True)
            return carry

        lax.fori_loop(0, base + (wid < rem).astype(jnp.int32), body, 0)
        plsc.subcore_barrier()
        pltpu.sync_copy(
            acc_sp.at[pl.ds(sid * rpt, rpt)],
            out_hbm.at[cid, pl.ds(sid * rpt, rpt)],
        )

    return scat_k


def _hprime_body(x_ref, w_ref, deg_ref, o_ref):
    dinv = lax.rsqrt(deg_ref[...] + 1.0)
    dinv = jnp.reshape(dinv, (NPAD, 1))[0:N]
    h = jnp.dot(x_ref[...], w_ref[...], preferred_element_type=jnp.float32)
    o_ref[...] = h * dinv


_hprime = pl.pallas_call(
    _hprime_body, out_shape=jax.ShapeDtypeStruct((N, D), jnp.float32)
)


def _final_body(prev_ref, a0_ref, a1_ref, hp_ref, deg_ref, b_ref, g_ref,
                be_ref, o_ref):
    dinv = lax.rsqrt(deg_ref[...] + 1.0)
    dinv = jnp.reshape(dinv, (NPAD, 1))[0:N]
    y = prev_ref[...] + dinv * (a0_ref[...] + a1_ref[...] + hp_ref[...])
    y = y + b_ref[...]
    mean = jnp.mean(y, axis=0, keepdims=True)
    c = y - mean
    var = jnp.mean(c * c, axis=0, keepdims=True)
    o_ref[...] = jnp.maximum(c * lax.rsqrt(var + EPS) * g_ref[...] + be_ref[...],
                             0.0)


_final = pl.pallas_call(
    _final_body, out_shape=jax.ShapeDtypeStruct((N, D), jnp.float32)
)


def kernel(prev, x, edge_index, W, b, gamma, beta):
    E = edge_index.shape[1]
    assert E % CHUNK == 0
    src = edge_index[0]
    dst = edge_index[1]

    ones_rows = jnp.ones((CHUNK, DEG_W), jnp.float32)
    zer_deg = jnp.zeros((NPAD // NS, DEG_W), jnp.float32)
    zer_acc = jnp.zeros((ZROWS, D), jnp.float32)

    degp = _make_deg(E)(dst, ones_rows, zer_deg)
    deg = (degp[0, :, 0] + degp[1, :, 0]).reshape(NPAD // 128, 128)

    hp = _hprime(x, W, deg)
    accp = _make_scatter(E)(hp, src, dst, zer_acc)

    return _final(
        prev, accp[0], accp[1], hp, deg,
        b.reshape(1, D), gamma.reshape(1, D), beta.reshape(1, D),
    )


# speedup vs baseline: 18.4402x; 18.4402x over previous
import functools

import jax
import jax.numpy as jnp
from jax import lax
from jax.experimental import pallas as pl
from jax.experimental.pallas import tpu as pltpu
from jax.experimental.pallas import tpu_sc as plsc

N = 10000
D = 128
EPS = 1e-5

NC = 2
NS = 16
NW = NC * NS

CHUNK = 128
NPAD = 10240
DEG_W = 128
ZROWS = 128

_mesh = plsc.VectorSubcoreMesh(
    core_axis_name="c", subcore_axis_name="s", num_cores=NC, num_subcores=NS
)


def _worker(cid, sid):
    return sid * NC + cid


def _make_deg(E):
    nchunk = E // CHUNK
    base, rem = nchunk // NW, nchunk % NW

    @functools.partial(
        pl.kernel,
        out_type=jax.ShapeDtypeStruct((NC, NPAD, DEG_W), jnp.float32),
        mesh=_mesh,
        scratch_types=[
            pltpu.VMEM((CHUNK,), jnp.int32),
            pltpu.VMEM((CHUNK, DEG_W), jnp.float32),
            pltpu.VMEM_SHARED((NPAD, DEG_W), jnp.float32),
            pltpu.SemaphoreType.DMA,
        ],
    )
    def deg_k(dst_hbm, ones_hbm, zer_hbm, out_hbm, idx_v, ones_v, deg_sp, sem):
        cid = lax.axis_index("c")
        sid = lax.axis_index("s")
        wid = _worker(cid, sid)
        rpt = NPAD // NS
        pltpu.sync_copy(zer_hbm, deg_sp.at[pl.ds(sid * rpt, rpt)])
        pltpu.sync_copy(ones_hbm, ones_v)
        plsc.subcore_barrier()

        def body(t, carry):
            j = wid + NW * t
            pltpu.sync_copy(dst_hbm.at[pl.ds(j * CHUNK, CHUNK)], idx_v)
            pltpu.sync_copy(ones_v, deg_sp.at[idx_v], add=True)
            return carry

        lax.fori_loop(0, base + (wid < rem).astype(jnp.int32), body, 0)
        plsc.subcore_barrier()
        pltpu.sync_copy(
            deg_sp.at[pl.ds(sid * rpt, rpt)],
            out_hbm.at[cid, pl.ds(sid * rpt, rpt)],
        )

    return deg_k


def _make_scatter(E):
    nchunk = E // CHUNK
    base, rem = nchunk // NW, nchunk % NW

    @functools.partial(
        pl.kernel,
        out_type=jax.ShapeDtypeStruct((NC, NPAD, D), jnp.float32),
        mesh=_mesh,
        scratch_types=[
            pltpu.VMEM((CHUNK,), jnp.int32),
            pltpu.VMEM((CHUNK,), jnp.int32),
            pltpu.VMEM((CHUNK, D), jnp.float32),
            pltpu.VMEM_SHARED((NPAD, D), jnp.float32),
            pltpu.SemaphoreType.DMA,
        ],
    )
    def scat_k(hp_hbm, src_hbm, dst_hbm, zer_hbm, out_hbm,
               sidx, didx, rows, acc_sp, sem):
        cid = lax.axis_index("c")
        sid = lax.axis_index("s")
        wid = _worker(cid, sid)
        rpt = NPAD // NS
        for z in range(rpt // ZROWS):
            pltpu.sync_copy(zer_hbm, acc_sp.at[pl.ds(sid * rpt + z * ZROWS, ZROWS)])
        plsc.subcore_barrier()

        def body(t, carry):
            j = wid + NW * t
            pltpu.sync_copy(src_hbm.at[pl.ds(j * CHUNK, CHUNK)], sidx)
            pltpu.sync_copy(dst_hbm.at[pl.ds(j * CHUNK, CHUNK)], didx)
            pltpu.async_copy(hp_hbm.at[sidx], rows, sem).wait()
            pltpu.sync_copy(rows, acc_sp.at[didx], add=True)
            return carry

        lax.fori_loop(0, base + (wid < rem).astype(jnp.int32), body, 0)
        plsc.subcore_barrier()
        pltpu.sync_copy(
            acc_sp.at[pl.ds(sid * rpt, rpt)],
            out_hbm.at[cid, pl.ds(sid * rpt, rpt)],
        )

    return scat_k


def _hprime_body(x_ref, w_ref, deg_ref, o_ref):
    dinv = lax.rsqrt(deg_ref[...] + 1.0)[0:N]
    h = jnp.dot(x_ref[...], w_ref[...], preferred_element_type=jnp.float32)
    o_ref[...] = h * dinv


_hprime = pl.pallas_call(
    _hprime_body, out_shape=jax.ShapeDtypeStruct((N, D), jnp.float32)
)


def _final_body(prev_ref, a0_ref, a1_ref, hp_ref, deg_ref, b_ref, g_ref,
                be_ref, o_ref):
    dinv = lax.rsqrt(deg_ref[...] + 1.0)[0:N]
    y = prev_ref[...] + dinv * (a0_ref[...] + a1_ref[...] + hp_ref[...])
    y = y + b_ref[...]
    mean = jnp.mean(y, axis=0, keepdims=True)
    c = y - mean
    var = jnp.mean(c * c, axis=0, keepdims=True)
    o_ref[...] = jnp.maximum(c * lax.rsqrt(var + EPS) * g_ref[...] + be_ref[...],
                             0.0)


_final = pl.pallas_call(
    _final_body, out_shape=jax.ShapeDtypeStruct((N, D), jnp.float32)
)


def kernel(prev, x, edge_index, W, b, gamma, beta):
    E = edge_index.shape[1]
    assert E % CHUNK == 0
    src = edge_index[0]
    dst = edge_index[1]

    ones_rows = jnp.ones((CHUNK, DEG_W), jnp.float32)
    zer_deg = jnp.zeros((NPAD // NS, DEG_W), jnp.float32)
    zer_acc = jnp.zeros((ZROWS, D), jnp.float32)

    degp = _make_deg(E)(dst, ones_rows, zer_deg)
    deg = (degp[0, :, 0] + degp[1, :, 0]).reshape(NPAD, 1)

    hp = _hprime(x, W, deg)
    accp = _make_scatter(E)(hp, src, dst, zer_acc)

    return _final(
        prev, accp[0, :N], accp[1, :N], hp, deg,
        b.reshape(1, D), gamma.reshape(1, D), beta.reshape(1, D),
    )


# speedup vs baseline: 28.5677x; 1.5492x over previous
import functools

import jax
import jax.numpy as jnp
from jax import lax
from jax.experimental import pallas as pl
from jax.experimental.pallas import tpu as pltpu
from jax.experimental.pallas import tpu_sc as plsc

N = 10000
D = 128
EPS = 1e-5

NC = 2
NS = 16
NW = NC * NS

CH = 128
EP = 327680
NROW = EP // CH
NCHT = NROW // NW
NPAD = 10240
ZROWS = 128
NBUF = 2
IRING = 8
DEG_B = 40

_mesh = plsc.VectorSubcoreMesh(
    core_axis_name="c", subcore_axis_name="s", num_cores=NC, num_subcores=NS
)


def _worker(cid, sid):
    return sid * NC + cid


def _make_deg():
    @functools.partial(
        pl.kernel,
        out_type=jax.ShapeDtypeStruct((NC, NPAD, D), jnp.float32),
        mesh=_mesh,
        scratch_types=[
            pltpu.VMEM((NCHT, CH), jnp.int32),
            pltpu.VMEM((CH, D), jnp.float32),
            pltpu.VMEM_SHARED((NPAD, D), jnp.float32),
            pltpu.SemaphoreType.DMA,
        ],
    )
    def deg_k(dst_hbm, ones_hbm, zer_hbm, out_hbm, didx_all, ones_v, deg_sp,
              sem):
        cid = lax.axis_index("c")
        sid = lax.axis_index("s")
        wid = _worker(cid, sid)
        rpt = NPAD // NS
        for z in range(rpt // ZROWS):
            pltpu.sync_copy(zer_hbm, deg_sp.at[pl.ds(sid * rpt + z * ZROWS, ZROWS)])
        pltpu.sync_copy(ones_hbm, ones_v)
        pltpu.sync_copy(dst_hbm.at[pl.ds(wid * NCHT, NCHT)], didx_all)
        plsc.subcore_barrier()

        def body(g, carry):
            for i in range(DEG_B):
                c = g * DEG_B + i
                pltpu.async_copy(ones_v, deg_sp.at[didx_all.at[c]], sem,
                                 add=True)
            for i in range(DEG_B):
                pltpu.make_async_copy(ones_v, deg_sp.at[didx_all.at[0]],
                                      sem).wait()
            return carry

        lax.fori_loop(0, NCHT // DEG_B, body, 0)
        plsc.subcore_barrier()
        pltpu.sync_copy(
            deg_sp.at[pl.ds(sid * rpt, rpt)],
            out_hbm.at[cid, pl.ds(sid * rpt, rpt)],
        )

    return deg_k


def _make_scatter():
    @functools.partial(
        pl.kernel,
        out_type=jax.ShapeDtypeStruct((NC, NPAD, D), jnp.float32),
        mesh=_mesh,
        scratch_types=[
            pltpu.VMEM((NCHT, CH), jnp.int32),
            pltpu.VMEM((IRING, CH), jnp.int32),
            pltpu.VMEM((NBUF, CH, D), jnp.float32),
            pltpu.VMEM_SHARED((NPAD, D), jnp.float32),
            pltpu.SemaphoreType.DMA((IRING,)),
            pltpu.SemaphoreType.DMA((NBUF,)),
            pltpu.SemaphoreType.DMA((NBUF,)),
        ],
    )
    def scat_k(hp_hbm, src_hbm, dst_hbm, zer_hbm, out_hbm,
               sidx_all, dring, rows, acc_sp, isem, gsem, ssem):
        cid = lax.axis_index("c")
        sid = lax.axis_index("s")
        wid = _worker(cid, sid)
        rpt = NPAD // NS
        for z in range(rpt // ZROWS):
            pltpu.sync_copy(zer_hbm, acc_sp.at[pl.ds(sid * rpt + z * ZROWS, ZROWS)])
        pltpu.sync_copy(src_hbm.at[pl.ds(wid * NCHT, NCHT)], sidx_all)
        for s in range(IRING):
            pltpu.async_copy(dst_hbm.at[pl.ds(wid * NCHT + s, 1)],
                             dring.at[pl.ds(s, 1)], isem.at[s])
        plsc.subcore_barrier()
        pltpu.async_copy(hp_hbm.at[sidx_all.at[0]], rows.at[0], gsem.at[0])

        def sup(g, carry):
            for b in range(IRING):
                c = g * IRING + b
                rb = b % NBUF
                rb1 = (b + 1) % NBUF
                sl = (b + IRING - 1) % IRING

                pltpu.make_async_copy(hp_hbm.at[sidx_all.at[0]], rows.at[rb],
                                      gsem.at[rb]).wait()

                @pl.when(c >= 1)
                def _():
                    pltpu.make_async_copy(rows.at[rb1],
                                          acc_sp.at[dring.at[0]],
                                          ssem.at[rb1]).wait()
                    r = c + IRING - 1

                    @pl.when(r < NCHT)
                    def _():
                        pltpu.async_copy(dst_hbm.at[pl.ds(wid * NCHT + r, 1)],
                                         dring.at[pl.ds(sl, 1)], isem.at[sl])

                @pl.when(c + 1 < NCHT)
                def _():
                    pltpu.async_copy(hp_hbm.at[sidx_all.at[c + 1]],
                                     rows.at[rb1], gsem.at[rb1])

                pltpu.make_async_copy(dst_hbm.at[pl.ds(0, 1)],
                                      dring.at[pl.ds(b, 1)], isem.at[b]).wait()
                pltpu.async_copy(rows.at[rb], acc_sp.at[dring.at[b]],
                                 ssem.at[rb], add=True)
            return carry

        lax.fori_loop(0, NCHT // IRING, sup, 0)
        pltpu.make_async_copy(rows.at[(NCHT - 1) % NBUF],
                              acc_sp.at[dring.at[0]],
                              ssem.at[(NCHT - 1) % NBUF]).wait()
        plsc.subcore_barrier()
        pltpu.sync_copy(
            acc_sp.at[pl.ds(sid * rpt, rpt)],
            out_hbm.at[cid, pl.ds(sid * rpt, rpt)],
        )

    return scat_k


def _hprime_body(x_ref, w_ref, degp_ref, o_ref):
    deg = degp_ref[0, 0:N, 0:1] + degp_ref[1, 0:N, 0:1]
    dinv = lax.rsqrt(deg + 1.0)
    h = jnp.dot(x_ref[...], w_ref[...], preferred_element_type=jnp.float32)
    o_ref[...] = h * dinv


_hprime = pl.pallas_call(
    _hprime_body, out_shape=jax.ShapeDtypeStruct((N, D), jnp.float32)
)


def _final_body(prev_ref, accp_ref, hp_ref, degp_ref, b_ref, g_ref,
                be_ref, o_ref):
    deg = degp_ref[0, 0:N, 0:1] + degp_ref[1, 0:N, 0:1]
    dinv = lax.rsqrt(deg + 1.0)
    a = accp_ref[0, 0:N, :] + accp_ref[1, 0:N, :]
    y = prev_ref[...] + dinv * (a + hp_ref[...])
    y = y + b_ref[...]
    mean = jnp.mean(y, axis=0, keepdims=True)
    c = y - mean
    var = jnp.mean(c * c, axis=0, keepdims=True)
    o_ref[...] = jnp.maximum(c * lax.rsqrt(var + EPS) * g_ref[...] + be_ref[...],
                             0.0)


_final = pl.pallas_call(
    _final_body, out_shape=jax.ShapeDtypeStruct((N, D), jnp.float32)
)


def kernel(prev, x, edge_index, W, b, gamma, beta):
    E = edge_index.shape[1]
    assert E % NW == 0 and E // NW <= NCHT * CH
    ppw = NCHT * CH - E // NW
    pad_src = jnp.tile(jnp.arange(ppw, dtype=jnp.int32), (NW, 1))
    pad_dst = N + pad_src % (NPAD - N)
    src = jnp.concatenate(
        [edge_index[0].reshape(NW, E // NW), pad_src], axis=1).reshape(NROW, CH)
    dst = jnp.concatenate(
        [edge_index[1].reshape(NW, E // NW), pad_dst], axis=1).reshape(NROW, CH)

    ones_rows = jnp.ones((CH, D), jnp.float32)
    zer = jnp.zeros((ZROWS, D), jnp.float32)

    degp = _make_deg()(dst, ones_rows, zer)
    hp = _hprime(x, W, degp)
    accp = _make_scatter()(hp, src, dst, zer)

    return _final(
        prev, accp, hp, degp,
        b.reshape(1, D), gamma.reshape(1, D), beta.reshape(1, D),
    )
